# baseline (speedup 1.0000x reference)
import jax
import jax.numpy as jnp
from jax.experimental import pallas as pl
from jax.experimental.pallas import tpu as pltpu
from functools import partial

NF_ = 64
H_ = 3
B_, N_, R_ = 2, 1024, 16384
PSTEP_ = 3
TR_ = 512
RT_ = R_ // TR_


def _relu(x):
    return jnp.maximum(x, 0.0)


def _dot(x, w):
    return jnp.dot(x, w, preferred_element_type=jnp.float32)


def _body(Rr_ref, Rs_ref, pe_in_ref, acs_ref,
          peW1_ref, peb1_ref, peW2_ref, peb2_ref,
          reA_ref, reC_ref, reb1_ref, reW2_ref, reb2_ref, reW3_ref, reb3_ref,
          rpWa_ref, rpb_ref, rpWb_ref, rpWc_ref,
          ppWa_ref, ppb_ref, ppWb_ref,
          prW1_ref, prb1_ref, prW2_ref, prb2_ref,
          out_ref,
          pf_scr, peproj_scr, agg_scr, pfb_scr, pfc_scr, rc_scr):
    p = pl.program_id(0)
    rt = pl.program_id(1)

    @pl.when((p == 0) & (rt == 0))
    def _():
        for b in range(B_):
            x = pe_in_ref[b]
            h = _relu(_dot(x, peW1_ref[...]) + peb1_ref[...])
            enc = _relu(_dot(h, peW2_ref[...]) + peb2_ref[...])
            pf_scr[b] = enc
            peproj_scr[b] = _dot(enc, ppWa_ref[...]) + ppb_ref[...]

    @pl.when(p == 0)
    def _():
        for b in range(B_):
            rr = Rr_ref[b].astype(jnp.bfloat16)
            rs = Rs_ref[b].astype(jnp.bfloat16)
            hr = _dot(rr, acs_ref[b].astype(jnp.bfloat16))
            hs = _dot(rs, acs_ref[b].astype(jnp.bfloat16))
            h1 = _relu(_dot(hr, reA_ref[...]) + _dot(hs, reC_ref[...])
                       + reb1_ref[...])
            h2 = _relu(_dot(h1, reW2_ref[...]) + reb2_ref[...])
            h3 = _relu(_dot(h2, reW3_ref[...]) + reb3_ref[...])
            rc_scr[b, pl.ds(rt * TR_, TR_), :] = (
                _dot(h3, rpWa_ref[...]) + rpb_ref[...])

    @pl.when(rt == 0)
    def _():
        agg_scr[...] = jnp.zeros_like(agg_scr)
        for b in range(B_):
            pfb_scr[b] = _dot(pf_scr[b], rpWb_ref[...]).astype(jnp.bfloat16)
            pfc_scr[b] = _dot(pf_scr[b], rpWc_ref[...]).astype(jnp.bfloat16)

    for b in range(B_):
        rr = Rr_ref[b].astype(jnp.bfloat16)
        rs = Rs_ref[b].astype(jnp.bfloat16)
        rel = _relu(rc_scr[b, pl.ds(rt * TR_, TR_), :]
                    + _dot(rr, pfb_scr[b]) + _dot(rs, pfc_scr[b]))
        agg_scr[b] += jax.lax.dot_general(
            rr, rel.astype(jnp.bfloat16),
            dimension_numbers=(((0,), (0,)), ((), ())),
            preferred_element_type=jnp.float32)

    @pl.when(rt == RT_ - 1)
    def _():
        for b in range(B_):
            pf_scr[b] = _relu(peproj_scr[b]
                              + _dot(agg_scr[b], ppWb_ref[...])
                              + pf_scr[b])

    @pl.when((p == PSTEP_ - 1) & (rt == RT_ - 1))
    def _():
        for b in range(B_):
            h = _relu(_dot(pf_scr[b], prW1_ref[...]) + prb1_ref[...])
            out_ref[b] = (_dot(h, prW2_ref[...]) + prb2_ref[...]
                          + acs_ref[b, :, 1:4])


@jax.jit
def kernel(a_hist, s_hist, s_delta, Rr, Rs,
           pe_W1, pe_b1, pe_W2, pe_b2,
           re_W1, re_b1, re_W2, re_b2, re_W3, re_b3,
           pp_W, pp_b, rp_W, rp_b,
           pr_W1, pr_b1, pr_W2, pr_b2):
    a = jnp.transpose(a_hist, (0, 2, 1))
    s = jnp.transpose(s_hist, (0, 2, 1, 3))
    sd = jnp.transpose(s_delta, (0, 2, 1, 3))
    sd_flat = sd.reshape(B_, N_, 3 * H_)
    pe_in = jnp.concatenate([sd_flat, a], axis=2)
    a_cur = a[:, :, -1]
    s_cur = s[:, :, -1, :]
    acs = jnp.concatenate([a_cur[..., None], s_cur], axis=2)

    reA = jnp.concatenate([re_W1[0:1], re_W1[2:5]], axis=0)
    reC = jnp.concatenate([re_W1[1:2], -re_W1[2:5]], axis=0)
    rpWa, rpWb, rpWc = rp_W[:NF_], rp_W[NF_:2 * NF_], rp_W[2 * NF_:]
    ppWa, ppWb = pp_W[:NF_], pp_W[NF_:]

    def row(v):
        return v.reshape(1, -1)

    full = lambda shape: pl.BlockSpec(shape, lambda p, rt: (0,) * len(shape))
    grid = (PSTEP_, RT_)

    in_specs = [
        pl.BlockSpec((B_, TR_, N_), lambda p, rt: (0, rt, 0)),
        pl.BlockSpec((B_, TR_, N_), lambda p, rt: (0, rt, 0)),
        full((B_, N_, 4 * H_)),
        full((B_, N_, 4)),
        full(pe_W1.shape), full((1, NF_)), full(pe_W2.shape), full((1, NF_)),
        full(reA.shape), full(reC.shape), full((1, NF_)),
        full(re_W2.shape), full((1, NF_)), full(re_W3.shape), full((1, NF_)),
        full(rpWa.shape), full((1, NF_)), full(rpWb.shape), full(rpWc.shape),
        full(ppWa.shape), full((1, NF_)), full(ppWb.shape),
        full(pr_W1.shape), full((1, NF_)), full(pr_W2.shape), full((1, 3)),
    ]

    pred = pl.pallas_call(
        _body,
        grid=grid,
        in_specs=in_specs,
        out_specs=full((B_, N_, 3)),
        out_shape=jax.ShapeDtypeStruct((B_, N_, 3), jnp.float32),
        scratch_shapes=[
            pltpu.VMEM((B_, N_, NF_), jnp.float32),
            pltpu.VMEM((B_, N_, NF_), jnp.float32),
            pltpu.VMEM((B_, N_, NF_), jnp.float32),
            pltpu.VMEM((B_, N_, NF_), jnp.bfloat16),
            pltpu.VMEM((B_, N_, NF_), jnp.bfloat16),
            pltpu.VMEM((B_, R_, NF_), jnp.float32),
        ],
        compiler_params=pltpu.CompilerParams(
            dimension_semantics=("arbitrary", "arbitrary"),
        ),
    )(Rr, Rs, pe_in, acs,
      pe_W1, row(pe_b1), pe_W2, row(pe_b2),
      reA, reC, row(re_b1), re_W2, row(re_b2), re_W3, row(re_b3),
      rpWa, row(rp_b), rpWb, rpWc,
      ppWa, row(pp_b), ppWb,
      pr_W1, row(pr_b1), pr_W2, row(pr_b2))
    return pred


# speedup vs baseline: 1.2678x; 1.2678x over previous
import jax
import jax.numpy as jnp
from jax.experimental import pallas as pl
from jax.experimental.pallas import tpu as pltpu

NF_ = 64
H_ = 3
B_, N_, R_ = 2, 1024, 16384
PSTEP_ = 3
TR_ = 512
RT_ = R_ // TR_
BF_ = jnp.bfloat16


def _relu(x):
    return jnp.maximum(x, 0.0)


def _dot(x, w):
    return jnp.dot(x, w, preferred_element_type=jnp.float32)


def _body_a(Rr_ref, Rs_ref, pe_in_ref, acs_ref,
            peW1_ref, peb1_ref, peW2_ref, peb2_ref,
            reA_ref, reC_ref, reb1_ref, reW2_ref, reb2_ref, reW3_ref,
            reb3_ref, rpWa_ref, rpb_ref, rpWb_ref, rpWc_ref,
            ppWa_ref, ppb_ref, ppWb_ref,
            Rr16_ref, Rs16_ref, rcp_ref, pf1_ref, peproj_ref,
            pf_scr, peproj_scr, agg_scr):
    rt = pl.program_id(0)

    @pl.when(rt == 0)
    def _():
        for b in range(B_):
            x = pe_in_ref[b]
            h = _relu(_dot(x, peW1_ref[...]) + peb1_ref[...])
            enc = _relu(_dot(h, peW2_ref[...]) + peb2_ref[...])
            pf_scr[b] = enc
            peproj_scr[b] = _dot(enc, ppWa_ref[...]) + ppb_ref[...]
        agg_scr[...] = jnp.zeros_like(agg_scr)

    for b in range(B_):
        rr = Rr_ref[b].astype(BF_)
        rs = Rs_ref[b].astype(BF_)
        Rr16_ref[b] = rr
        Rs16_ref[b] = rs
        hr = _dot(rr, acs_ref[b].astype(BF_))
        hs = _dot(rs, acs_ref[b].astype(BF_))
        h1 = _relu(_dot(hr, reA_ref[...]) + _dot(hs, reC_ref[...])
                   + reb1_ref[...])
        h2 = _relu(_dot(h1, reW2_ref[...]) + reb2_ref[...])
        h3 = _relu(_dot(h2, reW3_ref[...]) + reb3_ref[...])
        rcp = _dot(h3, rpWa_ref[...]) + rpb_ref[...]
        rcp_ref[b] = rcp
        pf = pf_scr[b].astype(BF_)
        er = _dot(rr, pf)
        es = _dot(rs, pf)
        rel = _relu(rcp + _dot(er, rpWb_ref[...]) + _dot(es, rpWc_ref[...]))
        agg_scr[b] += jax.lax.dot_general(
            rr, rel.astype(BF_),
            dimension_numbers=(((0,), (0,)), ((), ())),
            preferred_element_type=jnp.float32)

    @pl.when(rt == RT_ - 1)
    def _():
        for b in range(B_):
            pf1_ref[b] = _relu(peproj_scr[b]
                               + _dot(agg_scr[b], ppWb_ref[...])
                               + pf_scr[b])
            peproj_ref[b] = peproj_scr[b]


def _body_b(Rr16_ref, Rs16_ref, rcp_ref, pf1_ref, peproj_ref, acs_ref,
            rpWb_ref, rpWc_ref, ppWb_ref,
            prW1_ref, prb1_ref, prW2_ref, prb2_ref,
            out_ref,
            pf_scr, agg_scr):
    p = pl.program_id(0)
    rt = pl.program_id(1)

    @pl.when((p == 0) & (rt == 0))
    def _():
        pf_scr[...] = pf1_ref[...]

    @pl.when(rt == 0)
    def _():
        agg_scr[...] = jnp.zeros_like(agg_scr)

    for b in range(B_):
        rr = Rr16_ref[b]
        pf = pf_scr[b].astype(BF_)
        er = _dot(rr, pf)
        es = _dot(Rs16_ref[b], pf)
        rel = _relu(rcp_ref[b]
                    + _dot(er, rpWb_ref[...]) + _dot(es, rpWc_ref[...]))
        agg_scr[b] += jax.lax.dot_general(
            rr, rel.astype(BF_),
            dimension_numbers=(((0,), (0,)), ((), ())),
            preferred_element_type=jnp.float32)

    @pl.when(rt == RT_ - 1)
    def _():
        for b in range(B_):
            pf_scr[b] = _relu(peproj_ref[b]
                              + _dot(agg_scr[b], ppWb_ref[...])
                              + pf_scr[b])

    @pl.when((p == PSTEP_ - 2) & (rt == RT_ - 1))
    def _():
        for b in range(B_):
            h = _relu(_dot(pf_scr[b], prW1_ref[...]) + prb1_ref[...])
            out_ref[b] = (_dot(h, prW2_ref[...]) + prb2_ref[...]
                          + acs_ref[b, :, 1:4])


@jax.jit
def kernel(a_hist, s_hist, s_delta, Rr, Rs,
           pe_W1, pe_b1, pe_W2, pe_b2,
           re_W1, re_b1, re_W2, re_b2, re_W3, re_b3,
           pp_W, pp_b, rp_W, rp_b,
           pr_W1, pr_b1, pr_W2, pr_b2):
    a = jnp.transpose(a_hist, (0, 2, 1))
    s = jnp.transpose(s_hist, (0, 2, 1, 3))
    sd = jnp.transpose(s_delta, (0, 2, 1, 3))
    sd_flat = sd.reshape(B_, N_, 3 * H_)
    pe_in = jnp.concatenate([sd_flat, a], axis=2)
    a_cur = a[:, :, -1]
    s_cur = s[:, :, -1, :]
    acs = jnp.concatenate([a_cur[..., None], s_cur], axis=2)

    reA = jnp.concatenate([re_W1[0:1], re_W1[2:5]], axis=0)
    reC = jnp.concatenate([re_W1[1:2], -re_W1[2:5]], axis=0)
    rpWa, rpWb, rpWc = rp_W[:NF_], rp_W[NF_:2 * NF_], rp_W[2 * NF_:]
    ppWa, ppWb = pp_W[:NF_], pp_W[NF_:]

    def row(v):
        return v.reshape(1, -1)

    def tile_spec(nargs):
        return pl.BlockSpec((B_, TR_, N_), lambda *a: (0, a[-1], 0))

    def full(shape):
        return pl.BlockSpec(shape, lambda *a: (0,) * len(shape))

    in_specs_a = [
        tile_spec(1), tile_spec(1),
        full((B_, N_, 4 * H_)), full((B_, N_, 4)),
        full(pe_W1.shape), full((1, NF_)), full(pe_W2.shape), full((1, NF_)),
        full(reA.shape), full(reC.shape), full((1, NF_)),
        full(re_W2.shape), full((1, NF_)), full(re_W3.shape), full((1, NF_)),
        full(rpWa.shape), full((1, NF_)), full(rpWb.shape), full(rpWc.shape),
        full(ppWa.shape), full((1, NF_)), full(ppWb.shape),
    ]
    out_specs_a = [
        tile_spec(1), tile_spec(1),
        pl.BlockSpec((B_, TR_, NF_), lambda rt: (0, rt, 0)),
        full((B_, N_, NF_)), full((B_, N_, NF_)),
    ]
    out_shape_a = [
        jax.ShapeDtypeStruct((B_, R_, N_), BF_),
        jax.ShapeDtypeStruct((B_, R_, N_), BF_),
        jax.ShapeDtypeStruct((B_, R_, NF_), jnp.float32),
        jax.ShapeDtypeStruct((B_, N_, NF_), jnp.float32),
        jax.ShapeDtypeStruct((B_, N_, NF_), jnp.float32),
    ]
    Rr16, Rs16, rcp, pf1, peproj = pl.pallas_call(
        _body_a,
        grid=(RT_,),
        in_specs=in_specs_a,
        out_specs=out_specs_a,
        out_shape=out_shape_a,
        scratch_shapes=[
            pltpu.VMEM((B_, N_, NF_), jnp.float32),
            pltpu.VMEM((B_, N_, NF_), jnp.float32),
            pltpu.VMEM((B_, N_, NF_), jnp.float32),
        ],
        compiler_params=pltpu.CompilerParams(
            dimension_semantics=("arbitrary",),
        ),
    )(Rr, Rs, pe_in, acs,
      pe_W1, row(pe_b1), pe_W2, row(pe_b2),
      reA, reC, row(re_b1), re_W2, row(re_b2), re_W3, row(re_b3),
      rpWa, row(rp_b), rpWb, rpWc,
      ppWa, row(pp_b), ppWb)

    in_specs_b = [
        tile_spec(2), tile_spec(2),
        pl.BlockSpec((B_, TR_, NF_), lambda p, rt: (0, rt, 0)),
        full((B_, N_, NF_)), full((B_, N_, NF_)), full((B_, N_, 4)),
        full(rpWb.shape), full(rpWc.shape), full(ppWb.shape),
        full(pr_W1.shape), full((1, NF_)), full(pr_W2.shape), full((1, 3)),
    ]
    pred = pl.pallas_call(
        _body_b,
        grid=(PSTEP_ - 1, RT_),
        in_specs=in_specs_b,
        out_specs=full((B_, N_, 3)),
        out_shape=jax.ShapeDtypeStruct((B_, N_, 3), jnp.float32),
        scratch_shapes=[
            pltpu.VMEM((B_, N_, NF_), jnp.float32),
            pltpu.VMEM((B_, N_, NF_), jnp.float32),
        ],
        compiler_params=pltpu.CompilerParams(
            dimension_semantics=("arbitrary", "arbitrary"),
        ),
    )(Rr16, Rs16, rcp, pf1, peproj, acs,
      rpWb, rpWc, ppWb,
      pr_W1, row(pr_b1), pr_W2, row(pr_b2))
    return pred


# speedup vs baseline: 1.4865x; 1.1724x over previous
import jax
import jax.numpy as jnp
from jax.experimental import pallas as pl
from jax.experimental.pallas import tpu as pltpu

NF_ = 64
H_ = 3
B_, N_, R_ = 2, 1024, 16384
PSTEP_ = 3
TR_ = 512
RT_ = R_ // TR_
TRB_ = 1024
RTB_ = R_ // TRB_
BF_ = jnp.bfloat16


def _relu(x):
    return jnp.maximum(x, 0.0)


def _dot(x, w):
    return jnp.dot(x, w, preferred_element_type=jnp.float32)


def _body_a(Rr_ref, Rs_ref, pe_in_ref, acs_ref,
            peW1_ref, peb1_ref, peW2_ref, peb2_ref,
            reA_ref, reC_ref, reb1_ref, reW2_ref, reb2_ref, reW3_ref,
            reb3_ref, rpWa_ref, rpb_ref, rpWb_ref, rpWc_ref,
            ppWa_ref, ppb_ref, ppWb_ref,
            Rr16_ref, Rs16_ref, rcp_ref, pf1_ref, peproj_ref,
            pf_scr, peproj_scr, agg_scr, sa_scr, sc_scr):
    rt = pl.program_id(0)

    @pl.when(rt == 0)
    def _():
        for b in range(B_):
            x = pe_in_ref[b]
            h = _relu(_dot(x, peW1_ref[...]) + peb1_ref[...])
            enc = _relu(_dot(h, peW2_ref[...]) + peb2_ref[...])
            pf_scr[b] = enc
            peproj_scr[b] = _dot(enc, ppWa_ref[...]) + ppb_ref[...]
            acsA = _dot(acs_ref[b], reA_ref[...])
            acsC = _dot(acs_ref[b], reC_ref[...])
            enc16 = enc.astype(BF_)
            sa_scr[b] = jnp.concatenate([enc16, acsA.astype(BF_)], axis=1)
            sc_scr[b] = jnp.concatenate([enc16, acsC.astype(BF_)], axis=1)
        agg_scr[...] = jnp.zeros_like(agg_scr)

    for b in range(B_):
        rr = Rr_ref[b].astype(BF_)
        rs = Rs_ref[b].astype(BF_)
        Rr16_ref[b] = rr
        Rs16_ref[b] = rs
        cr = _dot(rr, sa_scr[b])
        cs = _dot(rs, sc_scr[b])
        er, hrA = cr[:, :NF_], cr[:, NF_:]
        es, hsC = cs[:, :NF_], cs[:, NF_:]
        h1 = _relu(hrA + hsC + reb1_ref[...])
        h2 = _relu(_dot(h1, reW2_ref[...]) + reb2_ref[...])
        h3 = _relu(_dot(h2, reW3_ref[...]) + reb3_ref[...])
        rcp = _dot(h3, rpWa_ref[...]) + rpb_ref[...]
        rcp_ref[b] = rcp
        rel = _relu(rcp + _dot(er, rpWb_ref[...]) + _dot(es, rpWc_ref[...]))
        agg_scr[b] += jax.lax.dot_general(
            rr, rel.astype(BF_),
            dimension_numbers=(((0,), (0,)), ((), ())),
            preferred_element_type=jnp.float32)

    @pl.when(rt == RT_ - 1)
    def _():
        for b in range(B_):
            pf1_ref[b] = _relu(peproj_scr[b]
                               + _dot(agg_scr[b], ppWb_ref[...])
                               + pf_scr[b])
            peproj_ref[b] = peproj_scr[b]


def _body_b(Rr16_ref, Rs16_ref, rcp_ref, pf1_ref, peproj_ref, acs_ref,
            rpWb_ref, rpWc_ref, ppWb_ref,
            prW1_ref, prb1_ref, prW2_ref, prb2_ref,
            out_ref,
            pf_scr, agg_scr):
    p = pl.program_id(0)
    rt = pl.program_id(1)

    @pl.when((p == 0) & (rt == 0))
    def _():
        pf_scr[...] = pf1_ref[...]

    @pl.when(rt == 0)
    def _():
        agg_scr[...] = jnp.zeros_like(agg_scr)

    for b in range(B_):
        rr = Rr16_ref[b]
        pf = pf_scr[b].astype(BF_)
        er = _dot(rr, pf)
        es = _dot(Rs16_ref[b], pf)
        rel = _relu(rcp_ref[b]
                    + _dot(er, rpWb_ref[...]) + _dot(es, rpWc_ref[...]))
        agg_scr[b] += jax.lax.dot_general(
            rr, rel.astype(BF_),
            dimension_numbers=(((0,), (0,)), ((), ())),
            preferred_element_type=jnp.float32)

    @pl.when(rt == RTB_ - 1)
    def _():
        for b in range(B_):
            pf_scr[b] = _relu(peproj_ref[b]
                              + _dot(agg_scr[b], ppWb_ref[...])
                              + pf_scr[b])

    @pl.when((p == PSTEP_ - 2) & (rt == RTB_ - 1))
    def _():
        for b in range(B_):
            h = _relu(_dot(pf_scr[b], prW1_ref[...]) + prb1_ref[...])
            out_ref[b] = (_dot(h, prW2_ref[...]) + prb2_ref[...]
                          + acs_ref[b, :, 1:4])


@jax.jit
def kernel(a_hist, s_hist, s_delta, Rr, Rs,
           pe_W1, pe_b1, pe_W2, pe_b2,
           re_W1, re_b1, re_W2, re_b2, re_W3, re_b3,
           pp_W, pp_b, rp_W, rp_b,
           pr_W1, pr_b1, pr_W2, pr_b2):
    a = jnp.transpose(a_hist, (0, 2, 1))
    s = jnp.transpose(s_hist, (0, 2, 1, 3))
    sd = jnp.transpose(s_delta, (0, 2, 1, 3))
    sd_flat = sd.reshape(B_, N_, 3 * H_)
    pe_in = jnp.concatenate([sd_flat, a], axis=2)
    a_cur = a[:, :, -1]
    s_cur = s[:, :, -1, :]
    acs = jnp.concatenate([a_cur[..., None], s_cur], axis=2)

    reA = jnp.concatenate([re_W1[0:1], re_W1[2:5]], axis=0)
    reC = jnp.concatenate([re_W1[1:2], -re_W1[2:5]], axis=0)
    rpWa, rpWb, rpWc = rp_W[:NF_], rp_W[NF_:2 * NF_], rp_W[2 * NF_:]
    ppWa, ppWb = pp_W[:NF_], pp_W[NF_:]

    def row(v):
        return v.reshape(1, -1)

    def tile_spec(nargs):
        return pl.BlockSpec((B_, TR_, N_), lambda *a: (0, a[-1], 0))

    def full(shape):
        return pl.BlockSpec(shape, lambda *a: (0,) * len(shape))

    in_specs_a = [
        tile_spec(1), tile_spec(1),
        full((B_, N_, 4 * H_)), full((B_, N_, 4)),
        full(pe_W1.shape), full((1, NF_)), full(pe_W2.shape), full((1, NF_)),
        full(reA.shape), full(reC.shape), full((1, NF_)),
        full(re_W2.shape), full((1, NF_)), full(re_W3.shape), full((1, NF_)),
        full(rpWa.shape), full((1, NF_)), full(rpWb.shape), full(rpWc.shape),
        full(ppWa.shape), full((1, NF_)), full(ppWb.shape),
    ]
    out_specs_a = [
        tile_spec(1), tile_spec(1),
        pl.BlockSpec((B_, TR_, NF_), lambda rt: (0, rt, 0)),
        full((B_, N_, NF_)), full((B_, N_, NF_)),
    ]
    out_shape_a = [
        jax.ShapeDtypeStruct((B_, R_, N_), BF_),
        jax.ShapeDtypeStruct((B_, R_, N_), BF_),
        jax.ShapeDtypeStruct((B_, R_, NF_), jnp.float32),
        jax.ShapeDtypeStruct((B_, N_, NF_), jnp.float32),
        jax.ShapeDtypeStruct((B_, N_, NF_), jnp.float32),
    ]
    Rr16, Rs16, rcp, pf1, peproj = pl.pallas_call(
        _body_a,
        grid=(RT_,),
        in_specs=in_specs_a,
        out_specs=out_specs_a,
        out_shape=out_shape_a,
        scratch_shapes=[
            pltpu.VMEM((B_, N_, NF_), jnp.float32),
            pltpu.VMEM((B_, N_, NF_), jnp.float32),
            pltpu.VMEM((B_, N_, NF_), jnp.float32),
            pltpu.VMEM((B_, N_, 2 * NF_), BF_),
            pltpu.VMEM((B_, N_, 2 * NF_), BF_),
        ],
        compiler_params=pltpu.CompilerParams(
            dimension_semantics=("arbitrary",),
        ),
    )(Rr, Rs, pe_in, acs,
      pe_W1, row(pe_b1), pe_W2, row(pe_b2),
      reA, reC, row(re_b1), re_W2, row(re_b2), re_W3, row(re_b3),
      rpWa, row(rp_b), rpWb, rpWc,
      ppWa, row(pp_b), ppWb)

    in_specs_b = [
        pl.BlockSpec((B_, TRB_, N_), lambda p, rt: (0, rt, 0)),
        pl.BlockSpec((B_, TRB_, N_), lambda p, rt: (0, rt, 0)),
        pl.BlockSpec((B_, TRB_, NF_), lambda p, rt: (0, rt, 0)),
        full((B_, N_, NF_)), full((B_, N_, NF_)), full((B_, N_, 4)),
        full(rpWb.shape), full(rpWc.shape), full(ppWb.shape),
        full(pr_W1.shape), full((1, NF_)), full(pr_W2.shape), full((1, 3)),
    ]
    pred = pl.pallas_call(
        _body_b,
        grid=(PSTEP_ - 1, RTB_),
        in_specs=in_specs_b,
        out_specs=full((B_, N_, 3)),
        out_shape=jax.ShapeDtypeStruct((B_, N_, 3), jnp.float32),
        scratch_shapes=[
            pltpu.VMEM((B_, N_, NF_), jnp.float32),
            pltpu.VMEM((B_, N_, NF_), jnp.float32),
        ],
        compiler_params=pltpu.CompilerParams(
            dimension_semantics=("arbitrary", "arbitrary"),
        ),
    )(Rr16, Rs16, rcp, pf1, peproj, acs,
      rpWb, rpWc, ppWb,
      pr_W1, row(pr_b1), pr_W2, row(pr_b2))
    return pred


# speedup vs baseline: 1.5635x; 1.0518x over previous
import jax
import jax.numpy as jnp
from jax.experimental import pallas as pl
from jax.experimental.pallas import tpu as pltpu

NF_ = 64
H_ = 3
B_, N_, R_ = 2, 1024, 16384
PSTEP_ = 3
TR_ = 512
RT_ = R_ // TR_
TRB_ = 1024
RTB_ = R_ // TRB_
BF_ = jnp.bfloat16


def _relu(x):
    return jnp.maximum(x, 0.0)


def _dot(x, w):
    return jnp.dot(x, w, preferred_element_type=jnp.float32)


def _body_a(Rr_ref, Rs_ref, pe_in_ref, acs_ref,
            peW1_ref, peb1_ref, peW2_ref, peb2_ref,
            reA_ref, reC_ref, reb1_ref, reW2_ref, reb2_ref, reW3_ref,
            reb3_ref, rpWa_ref, rpb_ref, rpWb_ref, rpWc_ref,
            ppWa_ref, ppb_ref, ppWb_ref,
            Rr16_ref, Rs16_ref, rcp_ref, pf1_ref, peproj_ref,
            pf_scr, peproj_scr, agg_scr, sa_scr, sc_scr):
    rt = pl.program_id(0)

    @pl.when(rt == 0)
    def _():
        for b in range(B_):
            x = pe_in_ref[b]
            h = _relu(_dot(x, peW1_ref[...]) + peb1_ref[...])
            enc = _relu(_dot(h, peW2_ref[...]) + peb2_ref[...])
            pf_scr[b] = enc
            peproj_scr[b] = _dot(enc, ppWa_ref[...]) + ppb_ref[...]
            acsA = _dot(acs_ref[b], reA_ref[...])
            acsC = _dot(acs_ref[b], reC_ref[...])
            enc16 = enc.astype(BF_)
            sa_scr[b] = jnp.concatenate([enc16, acsA.astype(BF_)], axis=1)
            sc_scr[b] = jnp.concatenate([enc16, acsC.astype(BF_)], axis=1)
        agg_scr[...] = jnp.zeros_like(agg_scr)

    rrs, crs = [], []
    for b in range(B_):
        rr = Rr_ref[b].astype(BF_)
        rs = Rs_ref[b].astype(BF_)
        Rr16_ref[b] = rr
        Rs16_ref[b] = rs
        rrs.append(rr)
        crs.append((_dot(rr, sa_scr[b]), _dot(rs, sc_scr[b])))
    rels = []
    for b in range(B_):
        cr, cs = crs[b]
        er, hrA = cr[:, :NF_], cr[:, NF_:]
        es, hsC = cs[:, :NF_], cs[:, NF_:]
        h1 = _relu(hrA + hsC + reb1_ref[...])
        h2 = _relu(_dot(h1, reW2_ref[...]) + reb2_ref[...])
        h3 = _relu(_dot(h2, reW3_ref[...]) + reb3_ref[...])
        rcp = _dot(h3, rpWa_ref[...]) + rpb_ref[...]
        rcp_ref[b] = rcp.astype(BF_)
        rel = _relu(rcp + _dot(er, rpWb_ref[...]) + _dot(es, rpWc_ref[...]))
        rels.append(rel.astype(BF_))
    for b in range(B_):
        agg_scr[b] += jax.lax.dot_general(
            rrs[b], rels[b],
            dimension_numbers=(((0,), (0,)), ((), ())),
            preferred_element_type=jnp.float32)

    @pl.when(rt == RT_ - 1)
    def _():
        for b in range(B_):
            pf1_ref[b] = _relu(peproj_scr[b]
                               + _dot(agg_scr[b], ppWb_ref[...])
                               + pf_scr[b])
            peproj_ref[b] = peproj_scr[b]


def _body_b(Rr16_ref, Rs16_ref, rcp_ref, pf1_ref, peproj_ref, acs_ref,
            rpWb_ref, rpWc_ref, ppWb_ref,
            prW1_ref, prb1_ref, prW2_ref, prb2_ref,
            out_ref,
            pf_scr, pf16_scr, agg_scr):
    p = pl.program_id(0)
    rt = pl.program_id(1)

    @pl.when((p == 0) & (rt == 0))
    def _():
        pf_scr[...] = pf1_ref[...]
        pf16_scr[...] = pf1_ref[...].astype(BF_)

    @pl.when(rt == 0)
    def _():
        agg_scr[...] = jnp.zeros_like(agg_scr)

    eres = []
    for b in range(B_):
        pf = pf16_scr[b]
        eres.append((_dot(Rr16_ref[b], pf), _dot(Rs16_ref[b], pf)))
    rels = []
    for b in range(B_):
        er, es = eres[b]
        rel = _relu(rcp_ref[b].astype(jnp.float32)
                    + _dot(er, rpWb_ref[...]) + _dot(es, rpWc_ref[...]))
        rels.append(rel.astype(BF_))
    for b in range(B_):
        agg_scr[b] += jax.lax.dot_general(
            Rr16_ref[b], rels[b],
            dimension_numbers=(((0,), (0,)), ((), ())),
            preferred_element_type=jnp.float32)

    @pl.when(rt == RTB_ - 1)
    def _():
        for b in range(B_):
            pf_new = _relu(peproj_ref[b]
                           + _dot(agg_scr[b], ppWb_ref[...])
                           + pf_scr[b])
            pf_scr[b] = pf_new
            pf16_scr[b] = pf_new.astype(BF_)

    @pl.when((p == PSTEP_ - 2) & (rt == RTB_ - 1))
    def _():
        for b in range(B_):
            h = _relu(_dot(pf_scr[b], prW1_ref[...]) + prb1_ref[...])
            out_ref[b] = (_dot(h, prW2_ref[...]) + prb2_ref[...]
                          + acs_ref[b, :, 1:4])


@jax.jit
def kernel(a_hist, s_hist, s_delta, Rr, Rs,
           pe_W1, pe_b1, pe_W2, pe_b2,
           re_W1, re_b1, re_W2, re_b2, re_W3, re_b3,
           pp_W, pp_b, rp_W, rp_b,
           pr_W1, pr_b1, pr_W2, pr_b2):
    a = jnp.transpose(a_hist, (0, 2, 1))
    s = jnp.transpose(s_hist, (0, 2, 1, 3))
    sd = jnp.transpose(s_delta, (0, 2, 1, 3))
    sd_flat = sd.reshape(B_, N_, 3 * H_)
    pe_in = jnp.concatenate([sd_flat, a], axis=2)
    a_cur = a[:, :, -1]
    s_cur = s[:, :, -1, :]
    acs = jnp.concatenate([a_cur[..., None], s_cur], axis=2)

    reA = jnp.concatenate([re_W1[0:1], re_W1[2:5]], axis=0)
    reC = jnp.concatenate([re_W1[1:2], -re_W1[2:5]], axis=0)
    rpWa, rpWb, rpWc = rp_W[:NF_], rp_W[NF_:2 * NF_], rp_W[2 * NF_:]
    ppWa, ppWb = pp_W[:NF_], pp_W[NF_:]

    def row(v):
        return v.reshape(1, -1)

    def tile_spec(nargs):
        return pl.BlockSpec((B_, TR_, N_), lambda *a: (0, a[-1], 0))

    def full(shape):
        return pl.BlockSpec(shape, lambda *a: (0,) * len(shape))

    in_specs_a = [
        tile_spec(1), tile_spec(1),
        full((B_, N_, 4 * H_)), full((B_, N_, 4)),
        full(pe_W1.shape), full((1, NF_)), full(pe_W2.shape), full((1, NF_)),
        full(reA.shape), full(reC.shape), full((1, NF_)),
        full(re_W2.shape), full((1, NF_)), full(re_W3.shape), full((1, NF_)),
        full(rpWa.shape), full((1, NF_)), full(rpWb.shape), full(rpWc.shape),
        full(ppWa.shape), full((1, NF_)), full(ppWb.shape),
    ]
    out_specs_a = [
        tile_spec(1), tile_spec(1),
        pl.BlockSpec((B_, TR_, NF_), lambda rt: (0, rt, 0)),
        full((B_, N_, NF_)), full((B_, N_, NF_)),
    ]
    out_shape_a = [
        jax.ShapeDtypeStruct((B_, R_, N_), BF_),
        jax.ShapeDtypeStruct((B_, R_, N_), BF_),
        jax.ShapeDtypeStruct((B_, R_, NF_), BF_),
        jax.ShapeDtypeStruct((B_, N_, NF_), jnp.float32),
        jax.ShapeDtypeStruct((B_, N_, NF_), jnp.float32),
    ]
    Rr16, Rs16, rcp, pf1, peproj = pl.pallas_call(
        _body_a,
        grid=(RT_,),
        in_specs=in_specs_a,
        out_specs=out_specs_a,
        out_shape=out_shape_a,
        scratch_shapes=[
            pltpu.VMEM((B_, N_, NF_), jnp.float32),
            pltpu.VMEM((B_, N_, NF_), jnp.float32),
            pltpu.VMEM((B_, N_, NF_), jnp.float32),
            pltpu.VMEM((B_, N_, 2 * NF_), BF_),
            pltpu.VMEM((B_, N_, 2 * NF_), BF_),
        ],
        compiler_params=pltpu.CompilerParams(
            dimension_semantics=("arbitrary",),
        ),
    )(Rr, Rs, pe_in, acs,
      pe_W1, row(pe_b1), pe_W2, row(pe_b2),
      reA, reC, row(re_b1), re_W2, row(re_b2), re_W3, row(re_b3),
      rpWa, row(rp_b), rpWb, rpWc,
      ppWa, row(pp_b), ppWb)

    in_specs_b = [
        pl.BlockSpec((B_, TRB_, N_), lambda p, rt: (0, rt, 0)),
        pl.BlockSpec((B_, TRB_, N_), lambda p, rt: (0, rt, 0)),
        pl.BlockSpec((B_, TRB_, NF_), lambda p, rt: (0, rt, 0)),
        full((B_, N_, NF_)), full((B_, N_, NF_)), full((B_, N_, 4)),
        full(rpWb.shape), full(rpWc.shape), full(ppWb.shape),
        full(pr_W1.shape), full((1, NF_)), full(pr_W2.shape), full((1, 3)),
    ]
    pred = pl.pallas_call(
        _body_b,
        grid=(PSTEP_ - 1, RTB_),
        in_specs=in_specs_b,
        out_specs=full((B_, N_, 3)),
        out_shape=jax.ShapeDtypeStruct((B_, N_, 3), jnp.float32),
        scratch_shapes=[
            pltpu.VMEM((B_, N_, NF_), jnp.float32),
            pltpu.VMEM((B_, N_, NF_), BF_),
            pltpu.VMEM((B_, N_, NF_), jnp.float32),
        ],
        compiler_params=pltpu.CompilerParams(
            dimension_semantics=("arbitrary", "arbitrary"),
        ),
    )(Rr16, Rs16, rcp, pf1, peproj, acs,
      rpWb, rpWc, ppWb,
      pr_W1, row(pr_b1), pr_W2, row(pr_b2))
    return pred


# speedup vs baseline: 1.6207x; 1.0366x over previous
import jax
import jax.numpy as jnp
from jax.experimental import pallas as pl
from jax.experimental.pallas import tpu as pltpu

NF_ = 64
H_ = 3
B_, N_, R_ = 2, 1024, 16384
PSTEP_ = 3
TR_ = 512
RT_ = R_ // TR_
TRB_ = 1024
RTB_ = R_ // TRB_
BF_ = jnp.bfloat16


def _relu(x):
    return jnp.maximum(x, 0.0)


def _dot(x, w):
    return jnp.dot(x, w, preferred_element_type=jnp.float32)


def _body_a(Rr_ref, Rs_ref, pe_in_ref, acs_ref,
            peW1_ref, peb1_ref, peW2_ref, peb2_ref,
            reA_ref, reC_ref, reb1_ref, reW2_ref, reb2_ref, reW3_ref,
            reb3_ref, rpWa_ref, rpb_ref, rpWb_ref, rpWc_ref,
            ppWa_ref, ppb_ref, ppWb_ref,
            Rr16_ref, Rs16_ref, rcp_ref, pf1_ref, peproj_ref,
            pf_scr, peproj_scr, agg_scr, sa_scr, sc_scr):
    rt = pl.program_id(0)

    @pl.when(rt == 0)
    def _():
        for b in range(B_):
            x = pe_in_ref[b]
            h = _relu(_dot(x, peW1_ref[...]) + peb1_ref[...])
            enc = _relu(_dot(h, peW2_ref[...]) + peb2_ref[...])
            pf_scr[b] = enc
            peproj_scr[b] = _dot(enc, ppWa_ref[...]) + ppb_ref[...]
            acsA = _dot(acs_ref[b], reA_ref[...])
            acsC = _dot(acs_ref[b], reC_ref[...])
            enc16 = enc.astype(BF_)
            sa_scr[b] = jnp.concatenate([enc16, acsA.astype(BF_)], axis=1)
            sc_scr[b] = jnp.concatenate([enc16, acsC.astype(BF_)], axis=1)
        agg_scr[...] = jnp.zeros_like(agg_scr)

    rrs, crs = [], []
    for b in range(B_):
        rr = Rr_ref[b].astype(BF_)
        rs = Rs_ref[b].astype(BF_)
        Rr16_ref[b] = rr
        Rs16_ref[b] = rs
        rrs.append(rr)
        crs.append((_dot(rr, sa_scr[b]), _dot(rs, sc_scr[b])))
    rels = []
    for b in range(B_):
        cr, cs = crs[b]
        er, hrA = cr[:, :NF_], cr[:, NF_:]
        es, hsC = cs[:, :NF_], cs[:, NF_:]
        h1 = _relu(hrA + hsC + reb1_ref[...])
        h2 = _relu(_dot(h1, reW2_ref[...]) + reb2_ref[...])
        h3 = _relu(_dot(h2, reW3_ref[...]) + reb3_ref[...])
        rcp = _dot(h3, rpWa_ref[...]) + rpb_ref[...]
        rcp_ref[b] = rcp.astype(BF_)
        rel = _relu(rcp + _dot(er, rpWb_ref[...]) + _dot(es, rpWc_ref[...]))
        rels.append(rel.astype(BF_))
    for b in range(B_):
        agg_scr[b] += jax.lax.dot_general(
            rels[b], rrs[b],
            dimension_numbers=(((0,), (0,)), ((), ())),
            preferred_element_type=jnp.float32)

    @pl.when(rt == RT_ - 1)
    def _():
        for b in range(B_):
            eff = jax.lax.dot_general(
                agg_scr[b], ppWb_ref[...],
                dimension_numbers=(((0,), (0,)), ((), ())),
                preferred_element_type=jnp.float32)
            pf1_ref[b] = _relu(peproj_scr[b] + eff + pf_scr[b])
            peproj_ref[b] = peproj_scr[b]


def _body_b(Rr16_ref, Rs16_ref, rcp_ref, pf1_ref, peproj_ref, acs_ref,
            rpWb_ref, rpWc_ref, ppWb_ref,
            prW1_ref, prb1_ref, prW2_ref, prb2_ref,
            out_ref,
            pf_scr, pf16_scr, agg_scr):
    p = pl.program_id(0)
    rt = pl.program_id(1)

    @pl.when((p == 0) & (rt == 0))
    def _():
        pf_scr[...] = pf1_ref[...]
        pf16_scr[...] = pf1_ref[...].astype(BF_)

    @pl.when(rt == 0)
    def _():
        agg_scr[...] = jnp.zeros_like(agg_scr)

    eres = []
    for b in range(B_):
        pf = pf16_scr[b]
        eres.append((_dot(Rr16_ref[b], pf), _dot(Rs16_ref[b], pf)))
    rels = []
    for b in range(B_):
        er, es = eres[b]
        rel = _relu(rcp_ref[b].astype(jnp.float32)
                    + _dot(er, rpWb_ref[...]) + _dot(es, rpWc_ref[...]))
        rels.append(rel.astype(BF_))
    for b in range(B_):
        agg_scr[b] += jax.lax.dot_general(
            rels[b], Rr16_ref[b],
            dimension_numbers=(((0,), (0,)), ((), ())),
            preferred_element_type=jnp.float32)

    @pl.when(rt == RTB_ - 1)
    def _():
        for b in range(B_):
            eff = jax.lax.dot_general(
                agg_scr[b], ppWb_ref[...],
                dimension_numbers=(((0,), (0,)), ((), ())),
                preferred_element_type=jnp.float32)
            pf_new = _relu(peproj_ref[b] + eff + pf_scr[b])
            pf_scr[b] = pf_new
            pf16_scr[b] = pf_new.astype(BF_)

    @pl.when((p == PSTEP_ - 2) & (rt == RTB_ - 1))
    def _():
        for b in range(B_):
            h = _relu(_dot(pf_scr[b], prW1_ref[...]) + prb1_ref[...])
            out_ref[b] = (_dot(h, prW2_ref[...]) + prb2_ref[...]
                          + acs_ref[b, :, 1:4])


@jax.jit
def kernel(a_hist, s_hist, s_delta, Rr, Rs,
           pe_W1, pe_b1, pe_W2, pe_b2,
           re_W1, re_b1, re_W2, re_b2, re_W3, re_b3,
           pp_W, pp_b, rp_W, rp_b,
           pr_W1, pr_b1, pr_W2, pr_b2):
    a = jnp.transpose(a_hist, (0, 2, 1))
    s = jnp.transpose(s_hist, (0, 2, 1, 3))
    sd = jnp.transpose(s_delta, (0, 2, 1, 3))
    sd_flat = sd.reshape(B_, N_, 3 * H_)
    pe_in = jnp.concatenate([sd_flat, a], axis=2)
    a_cur = a[:, :, -1]
    s_cur = s[:, :, -1, :]
    acs = jnp.concatenate([a_cur[..., None], s_cur], axis=2)

    reA = jnp.concatenate([re_W1[0:1], re_W1[2:5]], axis=0)
    reC = jnp.concatenate([re_W1[1:2], -re_W1[2:5]], axis=0)
    rpWa, rpWb, rpWc = rp_W[:NF_], rp_W[NF_:2 * NF_], rp_W[2 * NF_:]
    ppWa, ppWb = pp_W[:NF_], pp_W[NF_:]

    def row(v):
        return v.reshape(1, -1)

    def tile_spec(nargs):
        return pl.BlockSpec((B_, TR_, N_), lambda *a: (0, a[-1], 0))

    def full(shape):
        return pl.BlockSpec(shape, lambda *a: (0,) * len(shape))

    in_specs_a = [
        tile_spec(1), tile_spec(1),
        full((B_, N_, 4 * H_)), full((B_, N_, 4)),
        full(pe_W1.shape), full((1, NF_)), full(pe_W2.shape), full((1, NF_)),
        full(reA.shape), full(reC.shape), full((1, NF_)),
        full(re_W2.shape), full((1, NF_)), full(re_W3.shape), full((1, NF_)),
        full(rpWa.shape), full((1, NF_)), full(rpWb.shape), full(rpWc.shape),
        full(ppWa.shape), full((1, NF_)), full(ppWb.shape),
    ]
    out_specs_a = [
        tile_spec(1), tile_spec(1),
        pl.BlockSpec((B_, TR_, NF_), lambda rt: (0, rt, 0)),
        full((B_, N_, NF_)), full((B_, N_, NF_)),
    ]
    out_shape_a = [
        jax.ShapeDtypeStruct((B_, R_, N_), BF_),
        jax.ShapeDtypeStruct((B_, R_, N_), BF_),
        jax.ShapeDtypeStruct((B_, R_, NF_), BF_),
        jax.ShapeDtypeStruct((B_, N_, NF_), jnp.float32),
        jax.ShapeDtypeStruct((B_, N_, NF_), jnp.float32),
    ]
    Rr16, Rs16, rcp, pf1, peproj = pl.pallas_call(
        _body_a,
        grid=(RT_,),
        in_specs=in_specs_a,
        out_specs=out_specs_a,
        out_shape=out_shape_a,
        scratch_shapes=[
            pltpu.VMEM((B_, N_, NF_), jnp.float32),
            pltpu.VMEM((B_, N_, NF_), jnp.float32),
            pltpu.VMEM((B_, NF_, N_), jnp.float32),
            pltpu.VMEM((B_, N_, 2 * NF_), BF_),
            pltpu.VMEM((B_, N_, 2 * NF_), BF_),
        ],
        compiler_params=pltpu.CompilerParams(
            dimension_semantics=("arbitrary",),
        ),
    )(Rr, Rs, pe_in, acs,
      pe_W1, row(pe_b1), pe_W2, row(pe_b2),
      reA, reC, row(re_b1), re_W2, row(re_b2), re_W3, row(re_b3),
      rpWa, row(rp_b), rpWb, rpWc,
      ppWa, row(pp_b), ppWb)

    in_specs_b = [
        pl.BlockSpec((B_, TRB_, N_), lambda p, rt: (0, rt, 0)),
        pl.BlockSpec((B_, TRB_, N_), lambda p, rt: (0, rt, 0)),
        pl.BlockSpec((B_, TRB_, NF_), lambda p, rt: (0, rt, 0)),
        full((B_, N_, NF_)), full((B_, N_, NF_)), full((B_, N_, 4)),
        full(rpWb.shape), full(rpWc.shape), full(ppWb.shape),
        full(pr_W1.shape), full((1, NF_)), full(pr_W2.shape), full((1, 3)),
    ]
    pred = pl.pallas_call(
        _body_b,
        grid=(PSTEP_ - 1, RTB_),
        in_specs=in_specs_b,
        out_specs=full((B_, N_, 3)),
        out_shape=jax.ShapeDtypeStruct((B_, N_, 3), jnp.float32),
        scratch_shapes=[
            pltpu.VMEM((B_, N_, NF_), jnp.float32),
            pltpu.VMEM((B_, N_, NF_), BF_),
            pltpu.VMEM((B_, NF_, N_), jnp.float32),
        ],
        compiler_params=pltpu.CompilerParams(
            dimension_semantics=("arbitrary", "arbitrary"),
        ),
    )(Rr16, Rs16, rcp, pf1, peproj, acs,
      rpWb, rpWc, ppWb,
      pr_W1, row(pr_b1), pr_W2, row(pr_b2))
    return pred


# speedup vs baseline: 1.6518x; 1.0192x over previous
import jax
import jax.numpy as jnp
from jax.experimental import pallas as pl
from jax.experimental.pallas import tpu as pltpu

NF_ = 64
H_ = 3
B_, N_, R_ = 2, 1024, 16384
PSTEP_ = 3
TR_ = 512
RT_ = R_ // TR_
TRB_ = 2048
RTB_ = R_ // TRB_
BF_ = jnp.bfloat16


def _relu(x):
    return jnp.maximum(x, 0.0)


def _dot(x, w):
    return jnp.dot(x, w, preferred_element_type=jnp.float32)


def _body_a(Rr_ref, Rs_ref, pe_in_ref, acs_ref,
            peW1_ref, peb1_ref, peW2_ref, peb2_ref,
            reA_ref, reC_ref, reb1_ref, reW2_ref, reb2_ref, reW3_ref,
            reb3_ref, rpWa_ref, rpb_ref, rpWb_ref, rpWc_ref,
            ppWa_ref, ppb_ref, ppWb_ref,
            Rr16_ref, Rs16_ref, rcp_ref, pf1_ref, peproj_ref,
            pf_scr, peproj_scr, agg_scr, sa_scr, sc_scr):
    rt = pl.program_id(0)

    @pl.when(rt == 0)
    def _():
        for b in range(B_):
            x = pe_in_ref[b]
            h = _relu(_dot(x, peW1_ref[...]) + peb1_ref[...])
            enc = _relu(_dot(h, peW2_ref[...]) + peb2_ref[...])
            pf_scr[b] = enc
            peproj_scr[b] = _dot(enc, ppWa_ref[...]) + ppb_ref[...]
            acsA = _dot(acs_ref[b], reA_ref[...])
            acsC = _dot(acs_ref[b], reC_ref[...])
            enc16 = enc.astype(BF_)
            sa_scr[b] = jnp.concatenate([enc16, acsA.astype(BF_)], axis=1)
            sc_scr[b] = jnp.concatenate([enc16, acsC.astype(BF_)], axis=1)
        agg_scr[...] = jnp.zeros_like(agg_scr)

    rrs, crs = [], []
    for b in range(B_):
        rr = Rr_ref[b].astype(BF_)
        rs = Rs_ref[b].astype(BF_)
        Rr16_ref[b] = rr
        Rs16_ref[b] = rs
        rrs.append(rr)
        crs.append((_dot(rr, sa_scr[b]), _dot(rs, sc_scr[b])))
    rels = []
    for b in range(B_):
        cr, cs = crs[b]
        er, hrA = cr[:, :NF_], cr[:, NF_:]
        es, hsC = cs[:, :NF_], cs[:, NF_:]
        h1 = _relu(hrA + hsC + reb1_ref[...])
        h2 = _relu(_dot(h1, reW2_ref[...]) + reb2_ref[...])
        h3 = _relu(_dot(h2, reW3_ref[...]) + reb3_ref[...])
        rcp = _dot(h3, rpWa_ref[...]) + rpb_ref[...]
        rcp_ref[b] = rcp.astype(BF_)
        rel = _relu(rcp + _dot(er, rpWb_ref[...]) + _dot(es, rpWc_ref[...]))
        rels.append(rel.astype(BF_))
    for b in range(B_):
        agg_scr[b] += jax.lax.dot_general(
            rels[b], rrs[b],
            dimension_numbers=(((0,), (0,)), ((), ())),
            preferred_element_type=jnp.float32)

    @pl.when(rt == RT_ - 1)
    def _():
        for b in range(B_):
            eff = jax.lax.dot_general(
                agg_scr[b], ppWb_ref[...],
                dimension_numbers=(((0,), (0,)), ((), ())),
                preferred_element_type=jnp.float32)
            pf1_ref[b] = _relu(peproj_scr[b] + eff + pf_scr[b])
            peproj_ref[b] = peproj_scr[b]


def _body_b(Rr16_ref, Rs16_ref, rcp_ref, pf1_ref, peproj_ref, acs_ref,
            rpWb_ref, rpWc_ref, ppWb_ref,
            prW1_ref, prb1_ref, prW2_ref, prb2_ref,
            out_ref,
            pf_scr, pf16_scr, agg_scr):
    p = pl.program_id(0)
    rt = pl.program_id(1)

    @pl.when((p == 0) & (rt == 0))
    def _():
        pf_scr[...] = pf1_ref[...]
        pf16_scr[...] = pf1_ref[...].astype(BF_)

    @pl.when(rt == 0)
    def _():
        agg_scr[...] = jnp.zeros_like(agg_scr)

    eres = []
    for b in range(B_):
        pf = pf16_scr[b]
        eres.append((_dot(Rr16_ref[b], pf), _dot(Rs16_ref[b], pf)))
    rels = []
    for b in range(B_):
        er, es = eres[b]
        rel = _relu(rcp_ref[b].astype(jnp.float32)
                    + _dot(er, rpWb_ref[...]) + _dot(es, rpWc_ref[...]))
        rels.append(rel.astype(BF_))
    for b in range(B_):
        agg_scr[b] += jax.lax.dot_general(
            rels[b], Rr16_ref[b],
            dimension_numbers=(((0,), (0,)), ((), ())),
            preferred_element_type=jnp.float32)

    @pl.when(rt == RTB_ - 1)
    def _():
        for b in range(B_):
            eff = jax.lax.dot_general(
                agg_scr[b], ppWb_ref[...],
                dimension_numbers=(((0,), (0,)), ((), ())),
                preferred_element_type=jnp.float32)
            pf_new = _relu(peproj_ref[b] + eff + pf_scr[b])
            pf_scr[b] = pf_new
            pf16_scr[b] = pf_new.astype(BF_)

    @pl.when((p == PSTEP_ - 2) & (rt == RTB_ - 1))
    def _():
        for b in range(B_):
            h = _relu(_dot(pf_scr[b], prW1_ref[...]) + prb1_ref[...])
            out_ref[b] = (_dot(h, prW2_ref[...]) + prb2_ref[...]
                          + acs_ref[b, :, 1:4])


@jax.jit
def kernel(a_hist, s_hist, s_delta, Rr, Rs,
           pe_W1, pe_b1, pe_W2, pe_b2,
           re_W1, re_b1, re_W2, re_b2, re_W3, re_b3,
           pp_W, pp_b, rp_W, rp_b,
           pr_W1, pr_b1, pr_W2, pr_b2):
    a = jnp.transpose(a_hist, (0, 2, 1))
    s = jnp.transpose(s_hist, (0, 2, 1, 3))
    sd = jnp.transpose(s_delta, (0, 2, 1, 3))
    sd_flat = sd.reshape(B_, N_, 3 * H_)
    pe_in = jnp.concatenate([sd_flat, a], axis=2)
    a_cur = a[:, :, -1]
    s_cur = s[:, :, -1, :]
    acs = jnp.concatenate([a_cur[..., None], s_cur], axis=2)

    reA = jnp.concatenate([re_W1[0:1], re_W1[2:5]], axis=0)
    reC = jnp.concatenate([re_W1[1:2], -re_W1[2:5]], axis=0)
    rpWa, rpWb, rpWc = rp_W[:NF_], rp_W[NF_:2 * NF_], rp_W[2 * NF_:]
    ppWa, ppWb = pp_W[:NF_], pp_W[NF_:]

    def row(v):
        return v.reshape(1, -1)

    def tile_spec(nargs):
        return pl.BlockSpec((B_, TR_, N_), lambda *a: (0, a[-1], 0))

    def full(shape):
        return pl.BlockSpec(shape, lambda *a: (0,) * len(shape))

    in_specs_a = [
        tile_spec(1), tile_spec(1),
        full((B_, N_, 4 * H_)), full((B_, N_, 4)),
        full(pe_W1.shape), full((1, NF_)), full(pe_W2.shape), full((1, NF_)),
        full(reA.shape), full(reC.shape), full((1, NF_)),
        full(re_W2.shape), full((1, NF_)), full(re_W3.shape), full((1, NF_)),
        full(rpWa.shape), full((1, NF_)), full(rpWb.shape), full(rpWc.shape),
        full(ppWa.shape), full((1, NF_)), full(ppWb.shape),
    ]
    out_specs_a = [
        tile_spec(1), tile_spec(1),
        pl.BlockSpec((B_, TR_, NF_), lambda rt: (0, rt, 0)),
        full((B_, N_, NF_)), full((B_, N_, NF_)),
    ]
    out_shape_a = [
        jax.ShapeDtypeStruct((B_, R_, N_), BF_),
        jax.ShapeDtypeStruct((B_, R_, N_), BF_),
        jax.ShapeDtypeStruct((B_, R_, NF_), BF_),
        jax.ShapeDtypeStruct((B_, N_, NF_), jnp.float32),
        jax.ShapeDtypeStruct((B_, N_, NF_), jnp.float32),
    ]
    Rr16, Rs16, rcp, pf1, peproj = pl.pallas_call(
        _body_a,
        grid=(RT_,),
        in_specs=in_specs_a,
        out_specs=out_specs_a,
        out_shape=out_shape_a,
        scratch_shapes=[
            pltpu.VMEM((B_, N_, NF_), jnp.float32),
            pltpu.VMEM((B_, N_, NF_), jnp.float32),
            pltpu.VMEM((B_, NF_, N_), jnp.float32),
            pltpu.VMEM((B_, N_, 2 * NF_), BF_),
            pltpu.VMEM((B_, N_, 2 * NF_), BF_),
        ],
        compiler_params=pltpu.CompilerParams(
            dimension_semantics=("arbitrary",),
        ),
    )(Rr, Rs, pe_in, acs,
      pe_W1, row(pe_b1), pe_W2, row(pe_b2),
      reA, reC, row(re_b1), re_W2, row(re_b2), re_W3, row(re_b3),
      rpWa, row(rp_b), rpWb, rpWc,
      ppWa, row(pp_b), ppWb)

    in_specs_b = [
        pl.BlockSpec((B_, TRB_, N_), lambda p, rt: (0, rt, 0)),
        pl.BlockSpec((B_, TRB_, N_), lambda p, rt: (0, rt, 0)),
        pl.BlockSpec((B_, TRB_, NF_), lambda p, rt: (0, rt, 0)),
        full((B_, N_, NF_)), full((B_, N_, NF_)), full((B_, N_, 4)),
        full(rpWb.shape), full(rpWc.shape), full(ppWb.shape),
        full(pr_W1.shape), full((1, NF_)), full(pr_W2.shape), full((1, 3)),
    ]
    pred = pl.pallas_call(
        _body_b,
        grid=(PSTEP_ - 1, RTB_),
        in_specs=in_specs_b,
        out_specs=full((B_, N_, 3)),
        out_shape=jax.ShapeDtypeStruct((B_, N_, 3), jnp.float32),
        scratch_shapes=[
            pltpu.VMEM((B_, N_, NF_), jnp.float32),
            pltpu.VMEM((B_, N_, NF_), BF_),
            pltpu.VMEM((B_, NF_, N_), jnp.float32),
        ],
        compiler_params=pltpu.CompilerParams(
            dimension_semantics=("arbitrary", "arbitrary"),
        ),
    )(Rr16, Rs16, rcp, pf1, peproj, acs,
      rpWb, rpWc, ppWb,
      pr_W1, row(pr_b1), pr_W2, row(pr_b2))
    return pred
